# baseline (speedup 1.0000x reference)
import functools

import jax
import jax.numpy as jnp
from jax import lax
from jax.experimental import pallas as pl
from jax.experimental.pallas import tpu as pltpu
from jax.experimental.pallas import tpu_sc as plsc

D_MODEL = 64
BATCH = 4096
N_MOTIFS = 200

NC, NS = 2, 16
NW = NC * NS
B_PER_W = BATCH // NW
LANES = 16
MC = 4
N_CHUNK = N_MOTIFS // MC


@functools.partial(
    pl.kernel,
    out_type=jax.ShapeDtypeStruct((N_MOTIFS, D_MODEL, BATCH), jnp.float32),
    mesh=plsc.VectorSubcoreMesh(
        core_axis_name="c", subcore_axis_name="s",
        num_cores=NC, num_subcores=NS),
    scratch_types=[
        pltpu.VMEM((N_MOTIFS, B_PER_W), jnp.int32),
        pltpu.VMEM((2, D_MODEL, LANES), jnp.float32),
        pltpu.VMEM((2, MC, D_MODEL, B_PER_W), jnp.float32),
        pltpu.SemaphoreType.DMA,
        pltpu.SemaphoreType.DMA,
    ],
)
def _strand_encode(s_hbm, tab_hbm, out_hbm, s_v, t_v, out_v,
                   osem0, osem1):
    wid = lax.axis_index("s") * NC + lax.axis_index("c")
    b0 = wid * B_PER_W
    osems = (osem0, osem1)

    pltpu.sync_copy(s_hbm.at[:, pl.ds(b0, B_PER_W)], s_v)
    pltpu.sync_copy(tab_hbm, t_v)

    def out_slab(i):
        return out_hbm.at[pl.ds(i * MC, MC), :, pl.ds(b0, B_PER_W)]

    def half(ii, buf):
        i = 2 * ii + buf
        m0 = i * MC

        @pl.when(ii >= 1)
        def _():
            pltpu.make_async_copy(
                out_v.at[buf], out_slab(i - 2), osems[buf]).wait()

        @plsc.parallel_loop(0, D_MODEL, step=1, unroll=8)
        def _(d):
            e0 = t_v[0, d, :]
            dl = t_v[1, d, :] - e0
            for mm in range(MC):
                for j in range(B_PER_W // LANES):
                    sl = pl.ds(j * LANES, LANES)
                    sf = s_v[m0 + mm, sl].astype(jnp.float32)
                    out_v[buf, mm, d, sl] = e0 + sf * dl

        pltpu.async_copy(out_v.at[buf], out_slab(i), osems[buf])

    def chunk_pair(ii, carry):
        half(ii, 0)
        half(ii, 1)
        return carry

    lax.fori_loop(0, N_CHUNK // 2, chunk_pair, 0)
    for buf in range(2):
        pltpu.make_async_copy(
            out_v.at[buf], out_slab(N_CHUNK - 2 + buf), osems[buf]).wait()


def kernel(strands, strand_embed):
    s_t = strands.astype(jnp.int32).T
    tab = jnp.broadcast_to(
        strand_embed[:, :, None], (2, D_MODEL, LANES))
    out_t = _strand_encode(s_t, tab)
    return jnp.transpose(out_t, (2, 0, 1))


# speedup vs baseline: 1.0830x; 1.0830x over previous
import functools

import jax
import jax.numpy as jnp
from jax import lax
from jax.experimental import pallas as pl
from jax.experimental.pallas import tpu as pltpu
from jax.experimental.pallas import tpu_sc as plsc

D_MODEL = 64
BATCH = 4096
N_MOTIFS = 200

NC, NS = 2, 16
NW = NC * NS
B_PER_W = BATCH // NW
LANES = 16
MC = 4
N_CHUNK = N_MOTIFS // MC


@functools.partial(
    pl.kernel,
    out_type=jax.ShapeDtypeStruct((N_MOTIFS, D_MODEL, BATCH), jnp.float32),
    mesh=plsc.VectorSubcoreMesh(
        core_axis_name="c", subcore_axis_name="s",
        num_cores=NC, num_subcores=NS),
    scratch_types=[
        pltpu.VMEM((N_MOTIFS, B_PER_W), jnp.int32),
        pltpu.VMEM((2, D_MODEL, LANES), jnp.float32),
        pltpu.VMEM((MC, B_PER_W), jnp.float32),
        pltpu.VMEM((2, MC, D_MODEL, B_PER_W), jnp.float32),
        pltpu.SemaphoreType.DMA,
        pltpu.SemaphoreType.DMA,
    ],
)
def _strand_encode(s_hbm, tab_hbm, out_hbm, s_v, t_v, sf_v, out_v,
                   osem0, osem1):
    wid = lax.axis_index("s") * NC + lax.axis_index("c")
    b0 = wid * B_PER_W
    osems = (osem0, osem1)

    pltpu.sync_copy(s_hbm.at[:, pl.ds(b0, B_PER_W)], s_v)
    pltpu.sync_copy(tab_hbm, t_v)

    def out_slab(i):
        return out_hbm.at[pl.ds(i * MC, MC), :, pl.ds(b0, B_PER_W)]

    def half(ii, buf):
        i = 2 * ii + buf
        m0 = i * MC

        @pl.when(ii >= 1)
        def _():
            pltpu.make_async_copy(
                out_v.at[buf], out_slab(i - 2), osems[buf]).wait()

        for mm in range(MC):
            for j in range(B_PER_W // LANES):
                sl = pl.ds(j * LANES, LANES)
                sf_v[mm, sl] = s_v[m0 + mm, sl].astype(jnp.float32)

        @plsc.parallel_loop(0, D_MODEL, step=1, unroll=8)
        def _(d):
            e0 = t_v[0, d, :]
            dl = t_v[1, d, :] - e0
            for mm in range(MC):
                for j in range(B_PER_W // LANES):
                    sl = pl.ds(j * LANES, LANES)
                    out_v[buf, mm, d, sl] = e0 + sf_v[mm, sl] * dl

        pltpu.async_copy(out_v.at[buf], out_slab(i), osems[buf])

    def chunk_pair(ii, carry):
        half(ii, 0)
        half(ii, 1)
        return carry

    lax.fori_loop(0, N_CHUNK // 2, chunk_pair, 0)
    for buf in range(2):
        pltpu.make_async_copy(
            out_v.at[buf], out_slab(N_CHUNK - 2 + buf), osems[buf]).wait()


def kernel(strands, strand_embed):
    s_t = strands.astype(jnp.int32).T
    tab = jnp.broadcast_to(
        strand_embed[:, :, None], (2, D_MODEL, LANES))
    out_t = _strand_encode(s_t, tab)
    return jnp.transpose(out_t, (2, 0, 1))


# speedup vs baseline: 1.2527x; 1.1567x over previous
import functools

import jax
import jax.numpy as jnp
from jax import lax
from jax.experimental import pallas as pl
from jax.experimental.pallas import tpu as pltpu
from jax.experimental.pallas import tpu_sc as plsc

D_MODEL = 64
BATCH = 4096
N_MOTIFS = 200

NC, NS = 2, 16
NW = NC * NS
B_PER_W = BATCH // NW
LANES = 16
MC = 4
N_CHUNK = N_MOTIFS // MC


@functools.partial(
    pl.kernel,
    out_type=jax.ShapeDtypeStruct((N_MOTIFS, D_MODEL, BATCH), jnp.float32),
    mesh=plsc.VectorSubcoreMesh(
        core_axis_name="c", subcore_axis_name="s",
        num_cores=NC, num_subcores=NS),
    scratch_types=[
        pltpu.VMEM((N_MOTIFS, B_PER_W), jnp.float32),
        pltpu.VMEM((2, D_MODEL, LANES), jnp.float32),
        pltpu.VMEM((2, MC, D_MODEL, B_PER_W), jnp.float32),
        pltpu.SemaphoreType.DMA,
        pltpu.SemaphoreType.DMA,
    ],
)
def _strand_encode(s_hbm, tab_hbm, out_hbm, s_v, t_v, out_v,
                   osem0, osem1):
    wid = lax.axis_index("s") * NC + lax.axis_index("c")
    b0 = wid * B_PER_W
    osems = (osem0, osem1)

    pltpu.sync_copy(s_hbm.at[:, pl.ds(b0, B_PER_W)], s_v)
    pltpu.sync_copy(tab_hbm, t_v)

    def out_slab(i):
        return out_hbm.at[pl.ds(i * MC, MC), :, pl.ds(b0, B_PER_W)]

    def half(ii, buf):
        i = 2 * ii + buf
        m0 = i * MC

        @pl.when(ii >= 1)
        def _():
            pltpu.make_async_copy(
                out_v.at[buf], out_slab(i - 2), osems[buf]).wait()

        @plsc.parallel_loop(0, D_MODEL, step=1, unroll=4)
        def _(d):
            e0 = t_v[0, d, :]
            dl = t_v[1, d, :] - e0
            for mm in range(MC):
                for j in range(B_PER_W // LANES):
                    sl = pl.ds(j * LANES, LANES)
                    out_v[buf, mm, d, sl] = e0 + s_v[m0 + mm, sl] * dl

        pltpu.async_copy(out_v.at[buf], out_slab(i), osems[buf])

    def chunk_pair(ii, carry):
        half(ii, 0)
        half(ii, 1)
        return carry

    lax.fori_loop(0, N_CHUNK // 2, chunk_pair, 0)
    for buf in range(2):
        pltpu.make_async_copy(
            out_v.at[buf], out_slab(N_CHUNK - 2 + buf), osems[buf]).wait()


def kernel(strands, strand_embed):
    s_t = strands.astype(jnp.float32).T
    tab = jnp.broadcast_to(
        strand_embed[:, :, None], (2, D_MODEL, LANES))
    out_t = _strand_encode(s_t, tab)
    return jnp.transpose(out_t, (2, 0, 1))
